# Initial kernel scaffold; baseline (speedup 1.0000x reference)
import jax
import jax.numpy as jnp
from jax.experimental import pallas as pl


def kernel(x, W_in, conv_w, conv_b, A_log, D, dt_bias, W_out):
    raise NotImplementedError("write your pallas kernel here")


# speedup vs baseline: 2.6144x; 2.6144x over previous
import functools
import math

import jax
import jax.numpy as jnp
from jax.experimental import pallas as pl
from jax.experimental.pallas import tpu as pltpu

D_MODEL_C = 1024
D_STATE_C = 64
D_INNER_C = 2048
NHEADS_C = 32
HEADDIM_C = 64
D_CONV_C = 4

_T = 128


def _proj_main_kernel(x_ref, w_ref, o_ref):
    o_ref[...] = jnp.dot(x_ref[...], w_ref[...],
                         preferred_element_type=jnp.float32)


def _proj_main_call(x2d, w_t):
    M = x2d.shape[0]
    bm, bn = 256, 2048
    grid = (8192 // bn, M // bm)
    return pl.pallas_call(
        _proj_main_kernel,
        out_shape=jax.ShapeDtypeStruct((M, 8192), jnp.float32),
        grid=grid,
        in_specs=[
            pl.BlockSpec((bm, 1024), lambda j, i: (i, 0)),
            pl.BlockSpec((1024, bn), lambda j, i: (0, j)),
        ],
        out_specs=pl.BlockSpec((bm, bn), lambda j, i: (i, j)),
        compiler_params=pltpu.CompilerParams(
            dimension_semantics=("arbitrary", "arbitrary"),
        ),
        name="ssm_proj_main",
    )(x2d, w_t)


def _proj_dt_kernel(x_ref, w_ref, o_ref):
    o_ref[...] = jnp.dot(x_ref[...], w_ref[...],
                         precision=jax.lax.Precision.HIGHEST,
                         preferred_element_type=jnp.float32)


def _proj_dt_call(x2d, wdt_t):
    M = x2d.shape[0]
    bm = 512
    return pl.pallas_call(
        _proj_dt_kernel,
        out_shape=jax.ShapeDtypeStruct((M, 128), jnp.float32),
        grid=(M // bm,),
        in_specs=[
            pl.BlockSpec((bm, 1024), lambda i: (i, 0)),
            pl.BlockSpec((1024, 128), lambda i: (0, 0)),
        ],
        out_specs=pl.BlockSpec((bm, 128), lambda i: (i, 0)),
        name="ssm_proj_dt",
    )(x2d, wdt_t)


def _scan_kernel(xin_ref, z_ref, bp_ref, cp_ref, dtr_ref, convw_ref, convb_ref,
                 alog_ref, dtb_ref, dexp_ref, wo_ref, o_ref,
                 h_ref, tail_ref, da_ref, db_ref, cc_ref, xh_ref, yb_ref):
    c = pl.program_id(1)

    @pl.when(c == 0)
    def _():
        h_ref[...] = jnp.zeros_like(h_ref)
        tail_ref[...] = jnp.zeros_like(tail_ref)

    T = _T
    n, s, hd = NHEADS_C, D_STATE_C, HEADDIM_C

    xin = xin_ref[0]
    prev = tail_ref[...]
    ext = jnp.concatenate([prev[8 - (D_CONV_C - 1):], xin], axis=0)
    conv = convb_ref[...]
    acc = conv * jnp.ones((T, 1), jnp.float32)
    for k in range(D_CONV_C):
        acc = acc + convw_ref[k][None, :] * ext[k:k + T]
    tail_ref[...] = xin[T - 8:]
    xc = acc * jax.nn.sigmoid(acc)

    dt_raw = dtr_ref[0][:, :NHEADS_C] + dtb_ref[...]
    dt = jax.nn.softplus(dt_raw)
    a = -jnp.exp(alog_ref[...])

    da_ref[...] = jnp.exp(dt[:, :, None] * a[None, :, :])
    db_ref[...] = bp_ref[0].reshape(T, n, s) * dt[:, :, None]
    cc_ref[...] = cp_ref[0].reshape(T, n, s)
    xh_ref[...] = xc.reshape(T, n, hd)

    def step(t, _):
        hcur = da_ref[t][:, :, None] * h_ref[...] \
            + db_ref[t][:, :, None] * xh_ref[t][:, None, :]
        h_ref[...] = hcur
        yb_ref[t] = jnp.sum(cc_ref[t][:, :, None] * hcur, axis=1)
        return 0

    jax.lax.fori_loop(0, T, step, 0)

    y = yb_ref[...].reshape(T, n * hd) + dexp_ref[...] * xc
    z = z_ref[0]
    yg = y * (z * jax.nn.sigmoid(z))
    o_ref[0] = jnp.dot(yg, wo_ref[...], preferred_element_type=jnp.float32)


def _scan_call(xin, z, bp, cp, dtr, convw, convb, alog, dtb, dexp, wo_t):
    B, L = xin.shape[0], xin.shape[1]
    T = _T
    n, s, hd = NHEADS_C, D_STATE_C, HEADDIM_C
    grid = (B, L // T)
    specs3 = lambda lastdim: pl.BlockSpec((1, T, lastdim),
                                          lambda b, c: (b, c, 0))
    full2 = lambda shape: pl.BlockSpec(shape, lambda b, c: (0, 0))
    return pl.pallas_call(
        _scan_kernel,
        out_shape=jax.ShapeDtypeStruct((B, L, D_MODEL_C), jnp.float32),
        grid=grid,
        in_specs=[
            specs3(D_INNER_C),
            specs3(D_INNER_C),
            specs3(D_INNER_C),
            specs3(D_INNER_C),
            specs3(128),
            full2((D_CONV_C, D_INNER_C)),
            full2((1, D_INNER_C)),
            full2((NHEADS_C, D_STATE_C)),
            full2((1, NHEADS_C)),
            full2((1, D_INNER_C)),
            full2((D_INNER_C, D_MODEL_C)),
        ],
        out_specs=pl.BlockSpec((1, T, D_MODEL_C), lambda b, c: (b, c, 0)),
        scratch_shapes=[
            pltpu.VMEM((n, s, hd), jnp.float32),
            pltpu.VMEM((8, D_INNER_C), jnp.float32),
            pltpu.VMEM((T, n, s), jnp.float32),
            pltpu.VMEM((T, n, s), jnp.float32),
            pltpu.VMEM((T, n, s), jnp.float32),
            pltpu.VMEM((T, n, hd), jnp.float32),
            pltpu.VMEM((T, n, hd), jnp.float32),
        ],
        compiler_params=pltpu.CompilerParams(
            dimension_semantics=("parallel", "arbitrary"),
        ),
        name="ssm_scan_fused",
    )(xin, z, bp, cp, dtr, convw, convb, alog, dtb, dexp, wo_t)


@jax.jit
def kernel(x, W_in, conv_w, conv_b, A_log, D, dt_bias, W_out):
    B, L, dm = x.shape
    d, n, s = D_INNER_C, NHEADS_C, D_STATE_C

    W_x = W_in[:d]
    W_z = W_in[d:2 * d]
    tail = W_in[2 * d:].reshape(n, 2 * s + 1, dm)
    W_B = tail[:, :s].reshape(n * s, dm)
    W_C = tail[:, s:2 * s].reshape(n * s, dm)
    W_dt = tail[:, 2 * s]
    w_main_t = jnp.concatenate([W_x, W_z, W_B, W_C], axis=0).T
    w_dt_t = jnp.pad(W_dt, ((0, 128 - n), (0, 0))).T

    x2d = x.reshape(B * L, dm)
    proj = _proj_main_call(x2d, w_main_t)
    dtr = _proj_dt_call(x2d, w_dt_t)

    proj = proj.reshape(B, L, 8192)
    xin = proj[..., :d]
    z = proj[..., d:2 * d]
    bp = proj[..., 2 * d:2 * d + n * s]
    cp = proj[..., 2 * d + n * s:]
    dtr = dtr.reshape(B, L, 128)

    convw = conv_w[:, 0, :].T
    convb = conv_b.reshape(1, d)
    dtb = dt_bias.reshape(1, n)
    dexp = jnp.repeat(D, HEADDIM_C).reshape(1, d)
    wo_t = W_out.T

    return _scan_call(xin, z, bp, cp, dtr, convw, convb, A_log, dtb, dexp,
                      wo_t)


# speedup vs baseline: 2.8117x; 1.0755x over previous
import functools
import math

import jax
import jax.numpy as jnp
from jax.experimental import pallas as pl
from jax.experimental.pallas import tpu as pltpu

D_MODEL_C = 1024
D_STATE_C = 64
D_INNER_C = 2048
NHEADS_C = 32
HEADDIM_C = 64
D_CONV_C = 4

_T = 128
_TS = 16
_NSUB = _T // _TS


def _proj_main_kernel(x_ref, w_ref, o_ref):
    o_ref[...] = jnp.dot(x_ref[...], w_ref[...],
                         preferred_element_type=jnp.float32)


def _proj_main_call(x2d, w_t):
    M = x2d.shape[0]
    bm, bn = 256, 2048
    grid = (8192 // bn, M // bm)
    return pl.pallas_call(
        _proj_main_kernel,
        out_shape=jax.ShapeDtypeStruct((M, 8192), jnp.float32),
        grid=grid,
        in_specs=[
            pl.BlockSpec((bm, 1024), lambda j, i: (i, 0)),
            pl.BlockSpec((1024, bn), lambda j, i: (0, j)),
        ],
        out_specs=pl.BlockSpec((bm, bn), lambda j, i: (i, j)),
        compiler_params=pltpu.CompilerParams(
            dimension_semantics=("arbitrary", "arbitrary"),
        ),
        name="ssm_proj_main",
    )(x2d, w_t)


def _proj_dt_kernel(x_ref, w_ref, o_ref):
    o_ref[...] = jnp.dot(x_ref[...], w_ref[...],
                         precision=jax.lax.Precision.HIGHEST,
                         preferred_element_type=jnp.float32)


def _proj_dt_call(x2d, wdt_t):
    M = x2d.shape[0]
    bm = 512
    return pl.pallas_call(
        _proj_dt_kernel,
        out_shape=jax.ShapeDtypeStruct((M, 128), jnp.float32),
        grid=(M // bm,),
        in_specs=[
            pl.BlockSpec((bm, 1024), lambda i: (i, 0)),
            pl.BlockSpec((1024, 128), lambda i: (0, 0)),
        ],
        out_specs=pl.BlockSpec((bm, 128), lambda i: (i, 0)),
        name="ssm_proj_dt",
    )(x2d, wdt_t)


def _scan_kernel(xin_ref, z_ref, bp_ref, cp_ref, dtr_ref, convw_ref, convb_ref,
                 alog_ref, dtb_ref, dn_ref, wo_ref, o_ref,
                 h_ref, tail_ref, yb_ref):
    c_id = pl.program_id(1)

    @pl.when(c_id == 0)
    def _():
        h_ref[...] = jnp.zeros_like(h_ref)
        tail_ref[...] = jnp.zeros_like(tail_ref)

    T, ts, nsub = _T, _TS, _NSUB
    n, s, hd = NHEADS_C, D_STATE_C, HEADDIM_C

    xin = xin_ref[0]
    prev = tail_ref[...]
    ext = jnp.concatenate([prev[8 - (D_CONV_C - 1):], xin], axis=0)
    acc = convb_ref[...] * jnp.ones((T, 1), jnp.float32)
    for k in range(D_CONV_C):
        acc = acc + convw_ref[k][None, :] * ext[k:k + T]
    tail_ref[...] = xin[T - 8:]
    xc = acc * jax.nn.sigmoid(acc)

    dt = jax.nn.softplus(dtr_ref[0][:, :n] + dtb_ref[...])
    a = -jnp.exp(alog_ref[...])

    csum = dt
    k = 1
    while k < T:
        csum = csum + jnp.concatenate(
            [jnp.zeros((k, n), jnp.float32), csum[:T - k]], axis=0)
        k *= 2
    cT = csum.T
    dtT = dt.T

    bp4 = bp_ref[0]
    cp4 = cp_ref[0]
    dB = bp4 * dtT[:, :, None]
    xh = xc.reshape(T, n, hd).transpose(1, 0, 2)

    tri = (jax.lax.broadcasted_iota(jnp.int32, (1, ts, ts, s), 1)
           >= jax.lax.broadcasted_iota(jnp.int32, (1, ts, ts, s), 2))

    for j in range(nsub):
        sl = slice(j * ts, (j + 1) * ts)
        cl = cT[:, sl]
        dBj = dB[:, sl]
        Cj = cp4[:, sl]
        xj = xh[:, sl]
        cend = cT[:, j * ts + ts - 1][:, None]
        if j == 0:
            cstart = jnp.zeros((n, 1), jnp.float32)
        else:
            cstart = cT[:, j * ts - 1][:, None]

        delta = cl[:, :, None] - cl[:, None, :]
        arg = delta[:, :, :, None] * a[:, None, None, :]
        arg = jnp.where(tri, arg, -1e30)
        E = jnp.exp(arg)
        M = jnp.sum(Cj[:, :, None, :] * dBj[:, None, :, :] * E, axis=3)
        y0 = jnp.sum(M[:, :, :, None] * xj[:, None, :, :], axis=2)

        q = Cj * jnp.exp((cl[:, :, None] - cstart[:, :, None])
                         * a[:, None, :])
        hj = h_ref[...]
        y1 = jnp.sum(q[:, :, :, None] * hj[:, None, :, :], axis=2)

        yb_ref[:, sl, :] = y0 + y1

        K = dBj * jnp.exp((cend[:, :, None] - cl[:, :, None])
                          * a[:, None, :])
        Hj = jnp.sum(K[:, :, :, None] * xj[:, :, None, :], axis=1)
        P = jnp.exp(a * (cend - cstart))
        h_ref[...] = P[:, :, None] * hj + Hj

    y4 = yb_ref[...] + dn_ref[...][:, :, None] * xh
    y = y4.transpose(1, 0, 2).reshape(T, n * hd)
    z = z_ref[0]
    yg = y * (z * jax.nn.sigmoid(z))
    o_ref[0] = jnp.dot(yg, wo_ref[...], preferred_element_type=jnp.float32)


def _scan_call(xin, z, bp_t, cp_t, dtr, convw, convb, alog, dtb, dn, wo_t):
    B, L = xin.shape[0], xin.shape[1]
    T = _T
    n, s, hd = NHEADS_C, D_STATE_C, HEADDIM_C
    grid = (B, L // T)
    full = lambda shape: pl.BlockSpec(shape, lambda b, c: tuple(0 for _ in shape))
    return pl.pallas_call(
        _scan_kernel,
        out_shape=jax.ShapeDtypeStruct((B, L, D_MODEL_C), jnp.float32),
        grid=grid,
        in_specs=[
            pl.BlockSpec((1, T, D_INNER_C), lambda b, c: (b, c, 0)),
            pl.BlockSpec((1, T, D_INNER_C), lambda b, c: (b, c, 0)),
            pl.BlockSpec((1, n, T, s), lambda b, c: (b, 0, c, 0)),
            pl.BlockSpec((1, n, T, s), lambda b, c: (b, 0, c, 0)),
            pl.BlockSpec((1, T, 128), lambda b, c: (b, c, 0)),
            full((D_CONV_C, D_INNER_C)),
            full((1, D_INNER_C)),
            full((NHEADS_C, D_STATE_C)),
            full((1, NHEADS_C)),
            full((NHEADS_C, 1)),
            full((D_INNER_C, D_MODEL_C)),
        ],
        out_specs=pl.BlockSpec((1, T, D_MODEL_C), lambda b, c: (b, c, 0)),
        scratch_shapes=[
            pltpu.VMEM((n, s, hd), jnp.float32),
            pltpu.VMEM((8, D_INNER_C), jnp.float32),
            pltpu.VMEM((n, T, hd), jnp.float32),
        ],
        compiler_params=pltpu.CompilerParams(
            dimension_semantics=("parallel", "arbitrary"),
            vmem_limit_bytes=100 * 1024 * 1024,
        ),
        name="ssm_scan_fused",
    )(xin, z, bp_t, cp_t, dtr, convw, convb, alog, dtb, dn, wo_t)


@jax.jit
def kernel(x, W_in, conv_w, conv_b, A_log, D, dt_bias, W_out):
    B, L, dm = x.shape
    d, n, s = D_INNER_C, NHEADS_C, D_STATE_C

    W_x = W_in[:d]
    W_z = W_in[d:2 * d]
    tail = W_in[2 * d:].reshape(n, 2 * s + 1, dm)
    W_B = tail[:, :s].reshape(n * s, dm)
    W_C = tail[:, s:2 * s].reshape(n * s, dm)
    W_dt = tail[:, 2 * s]
    w_main_t = jnp.concatenate([W_x, W_z, W_B, W_C], axis=0).T
    w_dt_t = jnp.pad(W_dt, ((0, 128 - n), (0, 0))).T

    x2d = x.reshape(B * L, dm)
    proj = _proj_main_call(x2d, w_main_t)
    dtr = _proj_dt_call(x2d, w_dt_t)

    proj = proj.reshape(B, L, 8192)
    xin = proj[..., :d]
    z = proj[..., d:2 * d]
    bp_t = proj[..., 2 * d:2 * d + n * s].reshape(B, L, n, s).transpose(0, 2, 1, 3)
    cp_t = proj[..., 2 * d + n * s:].reshape(B, L, n, s).transpose(0, 2, 1, 3)
    dtr = dtr.reshape(B, L, 128)

    convw = conv_w[:, 0, :].T
    convb = conv_b.reshape(1, d)
    dtb = dt_bias.reshape(1, n)
    dn = D.reshape(n, 1)
    wo_t = W_out.T

    return _scan_call(xin, z, bp_t, cp_t, dtr, convw, convb, A_log, dtb, dn,
                      wo_t)


# speedup vs baseline: 9.9820x; 3.5501x over previous
import functools
import math

import jax
import jax.numpy as jnp
from jax.experimental import pallas as pl
from jax.experimental.pallas import tpu as pltpu

D_MODEL_C = 1024
D_STATE_C = 64
D_INNER_C = 2048
NHEADS_C = 32
HEADDIM_C = 64
D_CONV_C = 4

_T = 128
_TS = 16
_NSUB = _T // _TS

_HI = jax.lax.Precision.HIGHEST


def _proj_main_kernel(x_ref, w_ref, o_ref):
    o_ref[...] = jnp.dot(x_ref[...], w_ref[...],
                         preferred_element_type=jnp.float32)


def _proj_main_call(x2d, w_t):
    M = x2d.shape[0]
    bm, bn = 256, 2048
    grid = (8192 // bn, M // bm)
    return pl.pallas_call(
        _proj_main_kernel,
        out_shape=jax.ShapeDtypeStruct((M, 8192), jnp.float32),
        grid=grid,
        in_specs=[
            pl.BlockSpec((bm, 1024), lambda j, i: (i, 0)),
            pl.BlockSpec((1024, bn), lambda j, i: (0, j)),
        ],
        out_specs=pl.BlockSpec((bm, bn), lambda j, i: (i, j)),
        compiler_params=pltpu.CompilerParams(
            dimension_semantics=("arbitrary", "arbitrary"),
        ),
        name="ssm_proj_main",
    )(x2d, w_t)


def _proj_dt_kernel(x_ref, w_ref, o_ref):
    o_ref[...] = jnp.dot(x_ref[...], w_ref[...], precision=_HI,
                         preferred_element_type=jnp.float32)


def _proj_dt_call(x2d, wdt_t):
    M = x2d.shape[0]
    bm = 512
    return pl.pallas_call(
        _proj_dt_kernel,
        out_shape=jax.ShapeDtypeStruct((M, 128), jnp.float32),
        grid=(M // bm,),
        in_specs=[
            pl.BlockSpec((bm, 1024), lambda i: (i, 0)),
            pl.BlockSpec((1024, 128), lambda i: (0, 0)),
        ],
        out_specs=pl.BlockSpec((bm, 128), lambda i: (i, 0)),
        name="ssm_proj_dt",
    )(x2d, wdt_t)


def _scan_kernel(xin_ref, z_ref, bf_ref, cf_ref, bp4_ref, cp4_ref, dtr_ref,
                 convw_ref, convb_ref, alog_ref, dtb_ref, dexp_ref,
                 am_ref, ohup_ref, ohdn_ref, wo_ref, o_ref,
                 h_ref, tail_ref, yb4_ref, y0_ref):
    c_id = pl.program_id(1)

    @pl.when(c_id == 0)
    def _():
        h_ref[...] = jnp.zeros_like(h_ref)
        tail_ref[...] = jnp.zeros_like(tail_ref)

    T, ts, nsub = _T, _TS, _NSUB
    n, s, hd = NHEADS_C, D_STATE_C, HEADDIM_C

    xin = xin_ref[0]
    prev = tail_ref[...]
    ext = jnp.concatenate([prev[8 - (D_CONV_C - 1):], xin], axis=0)
    acc = convb_ref[...] * jnp.ones((T, 1), jnp.float32)
    for k in range(D_CONV_C):
        acc = acc + convw_ref[k][None, :] * ext[k:k + T]
    tail_ref[...] = xin[T - 8:]
    xc = acc * jax.nn.sigmoid(acc)

    dt = jax.nn.softplus(dtr_ref[0][:, :n] + dtb_ref[...])
    a = -jnp.exp(alog_ref[...])
    aT = a.T

    csum = dt
    k = 1
    while k < T:
        csum = csum + jnp.concatenate(
            [jnp.zeros((k, n), jnp.float32), csum[:T - k]], axis=0)
        k *= 2

    bf = jnp.dot(csum, am_ref[...], precision=_HI,
                 preferred_element_type=jnp.float32)
    dtf = jnp.dot(dt, ohup_ref[...], precision=_HI,
                  preferred_element_type=jnp.float32)
    dBf = bf_ref[0] * dtf
    cf = cf_ref[0]

    cT = csum.T
    dtT = dt.T
    dB4 = bp4_ref[0] * dtT[:, :, None]
    cp4 = cp4_ref[0]
    xh = xc.reshape(T, n, hd).transpose(1, 0, 2)

    i0 = jax.lax.broadcasted_iota(jnp.int32, (ts * ts, n * s), 0)
    mask = (i0 % ts) >= (i0 // ts)

    dims_qh = (((2,), (1,)), ((0,), (0,)))
    dims_kx = (((1,), (1,)), ((0,), (0,)))

    for j in range(nsub):
        sl = slice(j * ts, (j + 1) * ts)
        blj = bf[sl]
        rep_t = jnp.tile(blj, (ts, 1))
        rep_u = jnp.repeat(blj, ts, axis=0)
        e2 = jnp.exp(jnp.where(mask, rep_t - rep_u, -1e30))
        w2 = jnp.tile(cf[sl], (ts, 1)) * jnp.repeat(dBf[sl], ts, axis=0) * e2
        m2 = jnp.dot(w2, ohdn_ref[...],
                     preferred_element_type=jnp.float32)
        mx = jnp.dot(m2, ohup_ref[...],
                     preferred_element_type=jnp.float32)
        wy = mx * jnp.repeat(xc[sl], ts, axis=0)
        y0 = wy[0:ts]
        for u in range(1, ts):
            y0 = y0 + wy[u * ts:(u + 1) * ts]
        y0_ref[sl] = y0

        clj = cT[:, sl]
        cend = cT[:, j * ts + ts - 1][:, None]
        if j == 0:
            cstart = jnp.zeros((n, 1), jnp.float32)
        else:
            cstart = cT[:, j * ts - 1][:, None]
        q4 = cp4[:, sl] * jnp.exp((clj[:, :, None] - cstart[:, :, None])
                                  * a[:, None, :])
        hj = h_ref[...]
        yb4_ref[:, sl, :] = jax.lax.dot_general(
            q4, hj, dims_qh, preferred_element_type=jnp.float32)
        k4 = dB4[:, sl] * jnp.exp((cend[:, :, None] - clj[:, :, None])
                                  * a[:, None, :])
        hnew = jax.lax.dot_general(
            k4, xh[:, sl], dims_kx, preferred_element_type=jnp.float32)
        p4 = jnp.exp(a * (cend - cstart))
        h_ref[...] = p4[:, :, None] * hj + hnew

    y1 = yb4_ref[...].transpose(1, 0, 2).reshape(T, n * hd)
    y = y0_ref[...] + y1 + dexp_ref[...] * xc
    z = z_ref[0]
    yg = y * (z * jax.nn.sigmoid(z))
    o_ref[0] = jnp.dot(yg, wo_ref[...], preferred_element_type=jnp.float32)


def _scan_call(xin, z, bpf, cpf, bp4, cp4, dtr, convw, convb, alog, dtb,
               dexp, am, ohup, ohdn, wo_t):
    B, L = xin.shape[0], xin.shape[1]
    T = _T
    n, s, hd = NHEADS_C, D_STATE_C, HEADDIM_C
    grid = (B, L // T)
    full = lambda shape: pl.BlockSpec(shape, lambda b, c: tuple(0 for _ in shape))
    return pl.pallas_call(
        _scan_kernel,
        out_shape=jax.ShapeDtypeStruct((B, L, D_MODEL_C), jnp.float32),
        grid=grid,
        in_specs=[
            pl.BlockSpec((1, T, D_INNER_C), lambda b, c: (b, c, 0)),
            pl.BlockSpec((1, T, D_INNER_C), lambda b, c: (b, c, 0)),
            pl.BlockSpec((1, T, D_INNER_C), lambda b, c: (b, c, 0)),
            pl.BlockSpec((1, T, D_INNER_C), lambda b, c: (b, c, 0)),
            pl.BlockSpec((1, n, T, s), lambda b, c: (b, 0, c, 0)),
            pl.BlockSpec((1, n, T, s), lambda b, c: (b, 0, c, 0)),
            pl.BlockSpec((1, T, 128), lambda b, c: (b, c, 0)),
            full((D_CONV_C, D_INNER_C)),
            full((1, D_INNER_C)),
            full((NHEADS_C, D_STATE_C)),
            full((1, NHEADS_C)),
            full((1, D_INNER_C)),
            full((NHEADS_C, D_INNER_C)),
            full((NHEADS_C, D_INNER_C)),
            full((D_INNER_C, NHEADS_C)),
            full((D_INNER_C, D_MODEL_C)),
        ],
        out_specs=pl.BlockSpec((1, T, D_MODEL_C), lambda b, c: (b, c, 0)),
        scratch_shapes=[
            pltpu.VMEM((n, s, hd), jnp.float32),
            pltpu.VMEM((8, D_INNER_C), jnp.float32),
            pltpu.VMEM((n, T, hd), jnp.float32),
            pltpu.VMEM((T, D_INNER_C), jnp.float32),
        ],
        compiler_params=pltpu.CompilerParams(
            dimension_semantics=("parallel", "arbitrary"),
            vmem_limit_bytes=100 * 1024 * 1024,
        ),
        name="ssm_scan_fused",
    )(xin, z, bpf, cpf, bp4, cp4, dtr, convw, convb, alog, dtb, dexp,
      am, ohup, ohdn, wo_t)


@jax.jit
def kernel(x, W_in, conv_w, conv_b, A_log, D, dt_bias, W_out):
    B, L, dm = x.shape
    d, n, s = D_INNER_C, NHEADS_C, D_STATE_C

    W_x = W_in[:d]
    W_z = W_in[d:2 * d]
    tail = W_in[2 * d:].reshape(n, 2 * s + 1, dm)
    W_B = tail[:, :s].reshape(n * s, dm)
    W_C = tail[:, s:2 * s].reshape(n * s, dm)
    W_dt = tail[:, 2 * s]
    w_main_t = jnp.concatenate([W_x, W_z, W_B, W_C], axis=0).T
    w_dt_t = jnp.pad(W_dt, ((0, 128 - n), (0, 0))).T

    x2d = x.reshape(B * L, dm)
    proj = _proj_main_call(x2d, w_main_t)
    dtr = _proj_dt_call(x2d, w_dt_t)

    proj = proj.reshape(B, L, 8192)
    xin = proj[..., :d]
    z = proj[..., d:2 * d]
    bpf = proj[..., 2 * d:2 * d + n * s]
    cpf = proj[..., 2 * d + n * s:]
    bp4 = bpf.reshape(B, L, n, s).transpose(0, 2, 1, 3)
    cp4 = cpf.reshape(B, L, n, s).transpose(0, 2, 1, 3)
    dtr = dtr.reshape(B, L, 128)

    convw = conv_w[:, 0, :].T
    convb = conv_b.reshape(1, d)
    dtb = dt_bias.reshape(1, n)
    dexp = jnp.repeat(D, HEADDIM_C).reshape(1, d)
    wo_t = W_out.T

    a_neg = -jnp.exp(A_log)
    eye_n = jnp.eye(n, dtype=jnp.float32)
    am = (eye_n[:, :, None] * a_neg[None, :, :]).reshape(n, d)
    ohup = (eye_n[:, :, None]
            * jnp.ones((1, 1, s), jnp.float32)).reshape(n, d)
    ohdn = ohup.T

    return _scan_call(xin, z, bpf, cpf, bp4, cp4, dtr, convw, convb, A_log,
                      dtb, dexp, am, ohup, ohdn, wo_t)
